# Initial kernel scaffold; baseline (speedup 1.0000x reference)
import jax
import jax.numpy as jnp
from jax.experimental import pallas as pl


def kernel(node_feat, edge_feat, edge_index, W1, b1, Ws, bs, fcW, fcb):
    raise NotImplementedError("write your pallas kernel here")


# speedup vs baseline: 2.0801x; 2.0801x over previous
import functools

import jax
import jax.numpy as jnp
from jax import lax
from jax.experimental import pallas as pl
from jax.experimental.pallas import tpu as pltpu
from jax.experimental.pallas import tpu_sc as plsc

N = 50000
E = 800000
IN_FEATS = 16
H = 128
C = 4
N_HIDDEN_LAYERS = 19

NCORES = 2
NSUB = 16
SB = 512
SB_ROWS = SB // 128
E_PAD = 819200
PAD = E_PAD - E
NP = 50048
N_ACC = NP
ZROWS = N_ACC // NSUB
SB_PER_TILE_H = E_PAD // NSUB // SB
SB_PER_TILE_1 = E_PAD // (2 * NSUB) // SB

BN = 2000
GRID = N // BN


def _mesh():
    return plsc.VectorSubcoreMesh(
        core_axis_name="c", subcore_axis_name="s",
        num_cores=NCORES, num_subcores=NSUB)


def _deg_body(edges_hbm, zeros16_hbm, ones_hbm, out_hbm,
              acc, idxb, onesb, sem):
    c = lax.axis_index("c")
    t = lax.axis_index("s")
    pltpu.sync_copy(zeros16_hbm, acc.at[pl.ds(t * ZROWS, ZROWS), :])
    pltpu.sync_copy(ones_hbm, onesb)
    plsc.subcore_barrier()

    def body(sb, carry):
        row0 = (t * SB_PER_TILE_H + sb) * SB_ROWS
        pltpu.sync_copy(edges_hbm.at[c, pl.ds(row0, SB_ROWS)], idxb)
        for b in range(SB_ROWS):
            pltpu.sync_copy(onesb.at[b], acc.at[idxb.at[b]], add=True)
        return carry

    lax.fori_loop(0, SB_PER_TILE_H, body, 0)
    plsc.subcore_barrier()
    pltpu.sync_copy(acc.at[pl.ds(t * ZROWS, ZROWS), :],
                    out_hbm.at[pl.ds(c * N_ACC + t * ZROWS, ZROWS), :])


def _degrees(edges2, zeros16, ones_rows):
    k = pl.kernel(
        _deg_body,
        out_type=jax.ShapeDtypeStruct((2 * N_ACC, 16), jnp.float32),
        mesh=_mesh(),
        compiler_params=pltpu.CompilerParams(use_tc_tiling_on_sc=False),
        scratch_types=[
            pltpu.VMEM_SHARED((N_ACC, 16), jnp.float32),
            pltpu.VMEM((SB_ROWS, 128), jnp.int32),
            pltpu.VMEM((SB_ROWS, 128, 16), jnp.float32),
            pltpu.SemaphoreType.DMA,
        ],
    )
    return k(edges2, zeros16, ones_rows)


def _spmm_body(F, n_passes, split_edges,
               table_hbm, src_hbm, dst_hbm, ew_hbm, zeros_hbm, agg_hbm,
               acc, srcb, adjb, dstb, ewb, rowsb, sem):
    c = lax.axis_index("c")
    t = lax.axis_index("s")
    if split_edges:
        wid = c * NSUB + t
        sb_per = SB_PER_TILE_1
    else:
        wid = t
        sb_per = SB_PER_TILE_H

    for p in range(n_passes):
        chunk = NCORES * c + p if n_passes > 1 else c * 0
        off = chunk * NP
        pltpu.sync_copy(zeros_hbm, acc.at[pl.ds(t * ZROWS, ZROWS), :])
        plsc.subcore_barrier()

        def body(sb, carry):
            row0 = (wid * sb_per + sb) * SB_ROWS
            pltpu.sync_copy(src_hbm.at[pl.ds(row0, SB_ROWS)], srcb)
            pltpu.sync_copy(dst_hbm.at[pl.ds(row0, SB_ROWS)], dstb)
            pltpu.sync_copy(ew_hbm.at[pl.ds(row0, SB_ROWS)], ewb)
            for i in range(SB_ROWS):
                for j in range(8):
                    v = srcb[i, pl.ds(j * 16, 16)]
                    adjb[i, pl.ds(j * 16, 16)] = (
                        jnp.minimum(v, N - 1) + off)
            for b in range(SB_ROWS):
                pltpu.async_copy(table_hbm.at[adjb.at[b]], rowsb.at[b],
                                 sem).wait()

            def scale(g, carry2):
                i = g // 8
                j0 = pl.multiple_of((g % 8) * 16, 16)
                w16 = ewb[i, pl.ds(j0, 16)]
                for l in range(16):
                    w = w16[l]
                    for k2 in range(F // 16):
                        sl = rowsb[i, j0 + l, pl.ds(k2 * 16, 16)]
                        rowsb[i, j0 + l, pl.ds(k2 * 16, 16)] = sl * w
                return carry2

            lax.fori_loop(0, SB // 16, scale, 0)
            for b in range(SB_ROWS):
                pltpu.sync_copy(rowsb.at[b], acc.at[dstb.at[b]], add=True)
            return carry

        lax.fori_loop(0, sb_per, body, 0)
        plsc.subcore_barrier()
        if split_edges:
            out0 = c * NP
        else:
            out0 = off
        pltpu.sync_copy(acc.at[pl.ds(t * ZROWS, ZROWS), :],
                        agg_hbm.at[pl.ds(out0 + t * ZROWS, ZROWS), :])
        plsc.subcore_barrier()


def _spmm(table, src_r, dst_r, ew_r, zeros, F, n_passes, split_edges,
          out_rows):
    body = functools.partial(_spmm_body, F, n_passes, split_edges)
    k = pl.kernel(
        body,
        out_type=jax.ShapeDtypeStruct((out_rows, F), jnp.float32),
        mesh=_mesh(),
        compiler_params=pltpu.CompilerParams(use_tc_tiling_on_sc=False),
        scratch_types=[
            pltpu.VMEM_SHARED((N_ACC, F), jnp.float32),
            pltpu.VMEM((SB_ROWS, 128), jnp.int32),
            pltpu.VMEM((SB_ROWS, 128), jnp.int32),
            pltpu.VMEM((SB_ROWS, 128), jnp.int32),
            pltpu.VMEM((SB_ROWS, 128), jnp.float32),
            pltpu.VMEM((SB_ROWS, 128, F), jnp.float32),
            pltpu.SemaphoreType.DMA,
        ],
    )
    return k(table, src_r, dst_r, ew_r, zeros)


def _prologue_tc(deg2, node_feat):
    def body(deg_ref, nf_ref, ns_ref, nd_ref, t1_ref):
        dsrc = deg_ref[0][:, 0:1]
        ddst = deg_ref[1][:, 0:1]
        ns = lax.rsqrt(jnp.maximum(dsrc, 1.0))
        nd = lax.rsqrt(jnp.maximum(ddst, 1.0))
        ns_ref[...] = ns
        nd_ref[...] = nd
        t1_ref[...] = nf_ref[...] * ns

    return pl.pallas_call(
        body,
        grid=(GRID,),
        in_specs=[
            pl.BlockSpec((2, BN, 16), lambda i: (0, i, 0)),
            pl.BlockSpec((BN, IN_FEATS), lambda i: (i, 0)),
        ],
        out_specs=[
            pl.BlockSpec((BN, 1), lambda i: (i, 0)),
            pl.BlockSpec((BN, 1), lambda i: (i, 0)),
            pl.BlockSpec((BN, IN_FEATS), lambda i: (i, 0)),
        ],
        out_shape=[
            jax.ShapeDtypeStruct((N, 1), jnp.float32),
            jax.ShapeDtypeStruct((N, 1), jnp.float32),
            jax.ShapeDtypeStruct((NP, IN_FEATS), jnp.float32),
        ],
    )(deg2, node_feat)


def _layer_tc(agg, nd, ns, W, b, fin, fc_params=None):
    layer1 = agg.shape[0] == 2

    def body(agg_ref, nd_ref, ns_ref, w_ref, b_ref, *rest):
        if fc_params is None:
            out_ref = rest[0]
        else:
            fcw_ref, fcb_ref, out_ref = rest
        nd_ = nd_ref[...]
        if layer1:
            x = (agg_ref[0] + agg_ref[1]) * nd_
            y = jnp.dot(x, w_ref[...], preferred_element_type=jnp.float32)
        else:
            y = jnp.zeros((BN, H), jnp.float32)
            for cc in range(4):
                y = y + jnp.dot(agg_ref[cc] * nd_,
                                w_ref[0, cc * 32:(cc + 1) * 32, :],
                                preferred_element_type=jnp.float32)
        y = y + b_ref[...]
        h = jnp.where(y >= 0.0, y, 0.01 * y)
        if fc_params is None:
            hs = h * ns_ref[...]
            for cc in range(4):
                out_ref[cc] = hs[:, cc * 32:(cc + 1) * 32]
        else:
            out_ref[...] = jnp.dot(h, fcw_ref[...],
                                   preferred_element_type=jnp.float32)
            out_ref[...] += fcb_ref[...]

    kin = IN_FEATS if layer1 else H
    in_specs = [
        pl.BlockSpec(agg.shape[:1] + (BN, agg.shape[2]), lambda i: (0, i, 0)),
        pl.BlockSpec((BN, 1), lambda i: (i, 0)),
        pl.BlockSpec((BN, 1), lambda i: (i, 0)),
        pl.BlockSpec((kin, H) if layer1 else (1, kin, H),
                     (lambda i: (0, 0)) if layer1 else (lambda i: (0, 0, 0))),
        pl.BlockSpec((1, H), lambda i: (0, 0)),
    ]
    args = [agg, nd, ns, W if layer1 else W[None], b[None]]
    if fc_params is None:
        out_specs = pl.BlockSpec((4, BN, 32), lambda i: (0, i, 0))
        out_shape = jax.ShapeDtypeStruct((4, NP, 32), jnp.float32)
    else:
        fcW, fcb = fc_params
        in_specs += [
            pl.BlockSpec((H, C), lambda i: (0, 0)),
            pl.BlockSpec((1, C), lambda i: (0, 0)),
        ]
        args += [fcW, fcb[None]]
        out_specs = pl.BlockSpec((BN, C), lambda i: (i, 0))
        out_shape = jax.ShapeDtypeStruct((N, C), jnp.float32)

    return pl.pallas_call(
        body,
        grid=(GRID,),
        in_specs=in_specs,
        out_specs=out_specs,
        out_shape=out_shape,
    )(*args)


def kernel(node_feat, edge_feat, edge_index, W1, b1, Ws, bs, fcW, fcb):
    src = edge_index[0]
    dst = edge_index[1]
    ew = edge_feat[:, 0]

    padi = jnp.full((PAD,), N, jnp.int32)
    src_r = jnp.concatenate([src, padi]).reshape(E_PAD // 128, 128)
    dst_r = jnp.concatenate([dst, padi]).reshape(E_PAD // 128, 128)
    ew_r = jnp.concatenate([ew, jnp.zeros((PAD,), jnp.float32)]
                           ).reshape(E_PAD // 128, 128)

    zeros32 = jnp.zeros((ZROWS, 32), jnp.float32)
    zeros16 = jnp.zeros((ZROWS, IN_FEATS), jnp.float32)
    ones_rows = jnp.ones((SB_ROWS, 128, 16), jnp.float32)

    edges2 = jnp.stack([src_r, dst_r])
    deg = _degrees(edges2, zeros16, ones_rows)
    ns, nd, table1 = _prologue_tc(deg.reshape(2, N_ACC, 16), node_feat)

    agg1 = _spmm(table1, src_r, dst_r, ew_r, zeros16,
                 F=IN_FEATS, n_passes=1, split_edges=True, out_rows=2 * NP)
    table = _layer_tc(agg1.reshape(2, NP, IN_FEATS), nd, ns, W1, b1,
                      fin=False)

    for l in range(N_HIDDEN_LAYERS - 1):
        agg = _spmm(table.reshape(4 * NP, 32), src_r, dst_r, ew_r, zeros32,
                    F=32, n_passes=2, split_edges=False, out_rows=4 * NP)
        table = _layer_tc(agg.reshape(4, NP, 32), nd, ns, Ws[l], bs[l],
                          fin=False)

    agg = _spmm(table.reshape(4 * NP, 32), src_r, dst_r, ew_r, zeros32,
                F=32, n_passes=2, split_edges=False, out_rows=4 * NP)
    out = _layer_tc(agg.reshape(4, NP, 32), nd, ns,
                    Ws[N_HIDDEN_LAYERS - 1], bs[N_HIDDEN_LAYERS - 1],
                    fin=True, fc_params=(fcW, fcb))
    return out
